# baseline (speedup 1.0000x reference)
import functools

import jax
import jax.numpy as jnp
from jax import lax
from jax.experimental import pallas as pl
from jax.experimental.pallas import tpu as pltpu
from jax.experimental.pallas import tpu_sc as plsc

B = 16384
D = 32
H = 64
WIDE = 128
PACK = WIDE // D
NC = 2
NS = 16
NW = NC * NS
BPW = B // NW
L = 16


@functools.cache
def _make_sc_gather():
    mesh = plsc.VectorSubcoreMesh(core_axis_name="c", subcore_axis_name="s")

    @functools.partial(
        pl.kernel,
        mesh=mesh,
        out_type=jax.ShapeDtypeStruct((B, WIDE), jnp.float32),
        scratch_types=[
            pltpu.VMEM((BPW,), jnp.int32),
            pltpu.VMEM((BPW,), jnp.int32),
            pltpu.VMEM((BPW, WIDE), jnp.float32),
            pltpu.SemaphoreType.DMA,
        ],
    )
    def _sc_gather(idx_hbm, table_hbm, out_hbm, idx_v, idx4_v, rows_v, sem):
        wid = lax.axis_index("s") * NC + lax.axis_index("c")
        base = wid * BPW
        pltpu.sync_copy(idx_hbm.at[pl.ds(base, BPW)], idx_v)

        def body(i):
            sl = pl.ds(i * L, L)
            idx4_v[sl] = lax.shift_right_logical(idx_v[sl], 2)

        pl.loop(0, BPW // L)(body)
        pltpu.async_copy(table_hbm.at[idx4_v], rows_v, sem).wait()
        pltpu.sync_copy(rows_v, out_hbm.at[pl.ds(base, BPW)])

    return _sc_gather


def _mlp_body(wide_ref, uid_ref, bio_ref, mean_ref, var_ref, w1_ref, b1_ref,
              w2_ref, b2_ref, out_ref):
    inv = lax.rsqrt(var_ref[:] + 1e-7)
    xn = (bio_ref[:] - mean_ref[:]) * inv
    h = jnp.dot(xn, w1_ref[:], preferred_element_type=jnp.float32)
    h = jnp.maximum(h + b1_ref[:], 0.0)
    bio_vec = jnp.dot(h, w2_ref[:], preferred_element_type=jnp.float32)
    bio_vec = bio_vec + b2_ref[:]

    rem = uid_ref[:] & (PACK - 1)
    w = wide_ref[:]
    uv = w[:, 0 * D:1 * D]
    uv = jnp.where(rem == 1, w[:, 1 * D:2 * D], uv)
    uv = jnp.where(rem == 2, w[:, 2 * D:3 * D], uv)
    uv = jnp.where(rem == 3, w[:, 3 * D:4 * D], uv)
    out_ref[:] = jnp.concatenate([uv, bio_vec], axis=1)


_BLK = 2048


def _tc_mlp(wide, uid2, bio, mean2, var2, W1, b1_2, W2, b2_2):
    return pl.pallas_call(
        _mlp_body,
        grid=(B // _BLK,),
        in_specs=[
            pl.BlockSpec((_BLK, WIDE), lambda i: (i, 0)),
            pl.BlockSpec((_BLK, 1), lambda i: (i, 0)),
            pl.BlockSpec((_BLK, 3), lambda i: (i, 0)),
            pl.BlockSpec((1, 3), lambda i: (0, 0)),
            pl.BlockSpec((1, 3), lambda i: (0, 0)),
            pl.BlockSpec((3, H), lambda i: (0, 0)),
            pl.BlockSpec((1, H), lambda i: (0, 0)),
            pl.BlockSpec((H, D), lambda i: (0, 0)),
            pl.BlockSpec((1, D), lambda i: (0, 0)),
        ],
        out_specs=pl.BlockSpec((_BLK, 2 * D), lambda i: (i, 0)),
        out_shape=jax.ShapeDtypeStruct((B, 2 * D), jnp.float32),
    )(wide, uid2, bio, mean2, var2, W1, b1_2, W2, b2_2)


def kernel(user_id, P, E, I, emb_table, norm_mean, norm_var, W1, b1, W2, b2):
    table4 = emb_table.reshape(emb_table.shape[0] // PACK, WIDE)
    wide = _make_sc_gather()(user_id, table4)
    bio = jnp.stack([P, E, I], axis=1)
    return _tc_mlp(wide, user_id.reshape(B, 1), bio,
                   norm_mean.reshape(1, 3), norm_var.reshape(1, 3),
                   W1, b1.reshape(1, H), W2, b2.reshape(1, D))


# speedup vs baseline: 2.5702x; 2.5702x over previous
import functools

import jax
import jax.numpy as jnp
from jax import lax
from jax.experimental import pallas as pl
from jax.experimental.pallas import tpu as pltpu
from jax.experimental.pallas import tpu_sc as plsc

B = 16384
D = 32
H = 64
NU = 200000
NC = 2
NS = 16
CPS = D // NC
L = 16
CH = 102400
CHUNKS = ((0, 0, CH), (97280, CH, TAIL_LO := 199680))
TAIL = NU - TAIL_LO
NH = 2
BH = B // NH


@functools.cache
def _make_sc_gather():
    mesh = plsc.VectorSubcoreMesh(core_axis_name="c", subcore_axis_name="s")

    @functools.partial(
        pl.kernel,
        mesh=mesh,
        out_type=jax.ShapeDtypeStruct((D, B), jnp.float32),
        scratch_types=[
            pltpu.VMEM((CH,), jnp.float32),
            pltpu.VMEM((TAIL,), jnp.float32),
            pltpu.VMEM((BH,), jnp.int32),
            pltpu.VMEM((B,), jnp.float32),
            pltpu.SemaphoreType.DMA,
        ],
        compiler_params=pltpu.CompilerParams(needs_layout_passes=False),
    )
    def _sc_gather(uid_hbm, tableT_hbm, uvT_hbm, chunk_v, tail_v, uidh_v,
                   out_v, sem):
        cid = lax.axis_index("c")
        sid = lax.axis_index("s")
        c = cid * CPS + sid
        pltpu.sync_copy(tableT_hbm.at[c, pl.ds(TAIL_LO, TAIL)], tail_v)
        for p, (lo, mlo, mhi) in enumerate(CHUNKS):
            pltpu.sync_copy(tableT_hbm.at[c, pl.ds(lo, CH)], chunk_v)
            for hh in range(NH):
                hb = hh * BH
                pltpu.sync_copy(uid_hbm.at[pl.ds(hb, BH)], uidh_v)

                def it(i, p=p, hb=hb, lo=lo, mlo=mlo, mhi=mhi):
                    sl = pl.ds(i * L, L)
                    u = uidh_v[sl]
                    m = (u >= mlo) & (u < mhi)
                    lu = jnp.clip(u - lo, 0, CH - 1)
                    g = plsc.load_gather(chunk_v, [lu], mask=m)
                    osl = pl.ds(hb + i * L, L)
                    if p == 0:
                        mt = u >= TAIL_LO
                        lt = jnp.clip(u - TAIL_LO, 0, TAIL - 1)
                        gt = plsc.load_gather(tail_v, [lt], mask=mt)
                        out_v[osl] = jnp.where(
                            m, g, jnp.where(mt, gt, jnp.zeros_like(g)))
                    else:
                        out_v[osl] = jnp.where(m, g, out_v[osl])

                pl.loop(0, BH // L)(it)
        pltpu.sync_copy(out_v, uvT_hbm.at[c])

    return _sc_gather


def _mlp_body(uvT_ref, p_ref, e_ref, i_ref, mv_ref, w1t_ref, b1_ref,
              w2t_ref, b2_ref, out_ref):
    s0 = lax.rsqrt(mv_ref[3] + 1e-7)
    s1 = lax.rsqrt(mv_ref[4] + 1e-7)
    s2 = lax.rsqrt(mv_ref[5] + 1e-7)
    pn = (p_ref[:] - mv_ref[0]) * s0
    en = (e_ref[:] - mv_ref[1]) * s1
    inn = (i_ref[:] - mv_ref[2]) * s2
    hT = (w1t_ref[:, 0:1] * pn + w1t_ref[:, 1:2] * en
          + w1t_ref[:, 2:3] * inn + b1_ref[:])
    hT = jnp.maximum(hT, 0.0)
    bioT = jnp.dot(w2t_ref[:], hT, preferred_element_type=jnp.float32,
                   precision=lax.Precision.HIGHEST)
    out_ref[0:D, :] = uvT_ref[:]
    out_ref[D:2 * D, :] = bioT + b2_ref[:]


_BLK = 2048


def _tc_mlp(uvT, P, E, I, mv, W1T, b1c, W2T, b2c):
    return pl.pallas_call(
        _mlp_body,
        grid=(B // _BLK,),
        in_specs=[
            pl.BlockSpec((D, _BLK), lambda i: (0, i)),
            pl.BlockSpec((_BLK,), lambda i: (i,)),
            pl.BlockSpec((_BLK,), lambda i: (i,)),
            pl.BlockSpec((_BLK,), lambda i: (i,)),
            pl.BlockSpec(memory_space=pltpu.SMEM),
            pl.BlockSpec((H, 3), lambda i: (0, 0)),
            pl.BlockSpec((H, 1), lambda i: (0, 0)),
            pl.BlockSpec((D, H), lambda i: (0, 0)),
            pl.BlockSpec((D, 1), lambda i: (0, 0)),
        ],
        out_specs=pl.BlockSpec((2 * D, _BLK), lambda i: (0, i)),
        out_shape=jax.ShapeDtypeStruct((2 * D, B), jnp.float32),
    )(uvT, P, E, I, mv, W1T, b1c, W2T, b2c)


def kernel(user_id, P, E, I, emb_table, norm_mean, norm_var, W1, b1, W2, b2):
    tableT = emb_table.T
    uvT = _make_sc_gather()(user_id, tableT)
    mv = jnp.concatenate([norm_mean, norm_var])
    outT = _tc_mlp(uvT, P, E, I, mv,
                   W1.T, b1.reshape(H, 1), W2.T, b2.reshape(D, 1))
    return outT.T


# speedup vs baseline: 2.6584x; 1.0343x over previous
import functools

import jax
import jax.numpy as jnp
from jax import lax
from jax.experimental import pallas as pl
from jax.experimental.pallas import tpu as pltpu
from jax.experimental.pallas import tpu_sc as plsc

B = 16384
D = 32
H = 64
NU = 200000
NC = 2
NS = 16
CPS = D // NC
L = 16
CH = 102784
CHA = 102400
SPLIT = 102400
LO2 = 97280
TLO = 199680
NH = 2
BH = B // NH


@functools.cache
def _make_sc_gather():
    mesh = plsc.VectorSubcoreMesh(core_axis_name="c", subcore_axis_name="s")

    @functools.partial(
        pl.kernel,
        mesh=mesh,
        out_type=jax.ShapeDtypeStruct((D, B), jnp.float32),
        scratch_types=[
            pltpu.VMEM((CH,), jnp.float32),
            pltpu.VMEM((NU - TLO,), jnp.float32),
            pltpu.VMEM((BH,), jnp.int32),
            pltpu.VMEM((B,), jnp.float32),
            pltpu.SemaphoreType.DMA,
        ],
        compiler_params=pltpu.CompilerParams(needs_layout_passes=False),
    )
    def _sc_gather(uid_hbm, tableT_hbm, uvT_hbm, chunk_v, tail_v, uidh_v,
                   out_v, sem):
        cid = lax.axis_index("c")
        sid = lax.axis_index("s")
        c = cid * CPS + sid
        pltpu.sync_copy(tableT_hbm.at[c, pl.ds(TLO, NU - TLO)], tail_v)
        NSUB = 4
        SUB = CHA // NSUB
        for p, lo in enumerate((0, LO2)):
            copies = [
                pltpu.async_copy(
                    tableT_hbm.at[c, pl.ds(lo + s * SUB, SUB)],
                    chunk_v.at[pl.ds(s * SUB, SUB)], sem)
                for s in range(NSUB)
            ]
            for cp in copies:
                cp.wait()
            if p == 1:
                for k in range((NU - TLO) // L):
                    chunk_v[pl.ds(CHA + k * L, L)] = tail_v[pl.ds(k * L, L)]
            for hh in range(NH):
                hb = hh * BH
                pltpu.sync_copy(uid_hbm.at[pl.ds(hb, BH)], uidh_v)

                UNR = 4

                def it(i, p=p, hb=hb, lo=lo):
                    for j in range(UNR):
                        sl = pl.ds((i * UNR + j) * L, L)
                        u = uidh_v[sl]
                        osl = pl.ds(hb + (i * UNR + j) * L, L)
                        if p == 0:
                            m = u < SPLIT
                            g = plsc.load_gather(chunk_v, [u], mask=m)
                            out_v[osl] = jnp.where(m, g, jnp.zeros_like(g))
                        else:
                            m = u >= SPLIT
                            g = plsc.load_gather(chunk_v, [u - lo], mask=m)
                            out_v[osl] = jnp.where(m, g, out_v[osl])

                pl.loop(0, BH // (L * UNR))(it)
        pltpu.sync_copy(out_v, uvT_hbm.at[c])

    return _sc_gather


def _mlp_body(uvT_ref, p_ref, e_ref, i_ref, mv_ref, w1t_ref, b1_ref,
              w2t_ref, b2_ref, out_ref):
    s0 = lax.rsqrt(mv_ref[3] + 1e-7)
    s1 = lax.rsqrt(mv_ref[4] + 1e-7)
    s2 = lax.rsqrt(mv_ref[5] + 1e-7)
    pn = (p_ref[:] - mv_ref[0]) * s0
    en = (e_ref[:] - mv_ref[1]) * s1
    inn = (i_ref[:] - mv_ref[2]) * s2
    hT = (w1t_ref[:, 0:1] * pn + w1t_ref[:, 1:2] * en
          + w1t_ref[:, 2:3] * inn + b1_ref[:])
    hT = jnp.maximum(hT, 0.0)
    bioT = jnp.dot(w2t_ref[:], hT, preferred_element_type=jnp.float32)
    out_ref[0:D, :] = uvT_ref[:]
    out_ref[D:2 * D, :] = bioT + b2_ref[:]


_BLK = 2048


def _tc_mlp(uvT, P, E, I, mv, W1T, b1c, W2T, b2c):
    return pl.pallas_call(
        _mlp_body,
        grid=(B // _BLK,),
        in_specs=[
            pl.BlockSpec((D, _BLK), lambda i: (0, i)),
            pl.BlockSpec((_BLK,), lambda i: (i,)),
            pl.BlockSpec((_BLK,), lambda i: (i,)),
            pl.BlockSpec((_BLK,), lambda i: (i,)),
            pl.BlockSpec(memory_space=pltpu.SMEM),
            pl.BlockSpec((H, 3), lambda i: (0, 0)),
            pl.BlockSpec((H, 1), lambda i: (0, 0)),
            pl.BlockSpec((D, H), lambda i: (0, 0)),
            pl.BlockSpec((D, 1), lambda i: (0, 0)),
        ],
        out_specs=pl.BlockSpec((2 * D, _BLK), lambda i: (0, i)),
        out_shape=jax.ShapeDtypeStruct((2 * D, B), jnp.float32),
    )(uvT, P, E, I, mv, W1T, b1c, W2T, b2c)


def kernel(user_id, P, E, I, emb_table, norm_mean, norm_var, W1, b1, W2, b2):
    tableT = emb_table.T
    uvT = _make_sc_gather()(user_id, tableT)
    mv = jnp.concatenate([norm_mean, norm_var])
    outT = _tc_mlp(uvT, P, E, I, mv,
                   W1.T, b1.reshape(H, 1), W2.T, b2.reshape(D, 1))
    return outT.T


# speedup vs baseline: 2.7332x; 1.0281x over previous
import functools

import jax
import jax.numpy as jnp
from jax import lax
from jax.experimental import pallas as pl
from jax.experimental.pallas import tpu as pltpu
from jax.experimental.pallas import tpu_sc as plsc

B = 16384
D = 32
H = 64
NU = 200000
NC = 2
NS = 16
CPS = D // NC
L = 16
CH = 102784
CHA = 102400
SPLIT = 102400
LO2 = 97280
TLO = 199680
NH = 2
BH = B // NH


@functools.cache
def _make_sc_gather():
    mesh = plsc.VectorSubcoreMesh(core_axis_name="c", subcore_axis_name="s")

    @functools.partial(
        pl.kernel,
        mesh=mesh,
        out_type=jax.ShapeDtypeStruct((2 * D, B), jnp.float32),
        scratch_types=[
            pltpu.VMEM((CH,), jnp.float32),
            pltpu.VMEM((NU - TLO,), jnp.float32),
            pltpu.VMEM((BH,), jnp.int32),
            pltpu.VMEM((B,), jnp.float32),
            pltpu.SemaphoreType.DMA,
        ],
        compiler_params=pltpu.CompilerParams(needs_layout_passes=False),
    )
    def _sc_gather(uid_hbm, tableT_hbm, uvT_hbm, chunk_v, tail_v, uidh_v,
                   out_v, sem):
        cid = lax.axis_index("c")
        sid = lax.axis_index("s")
        c = cid * CPS + sid
        pltpu.sync_copy(tableT_hbm.at[c, pl.ds(TLO, NU - TLO)], tail_v)
        NSUB = 4
        SUB = CHA // NSUB
        for p, lo in enumerate((0, LO2)):
            copies = [
                pltpu.async_copy(
                    tableT_hbm.at[c, pl.ds(lo + s * SUB, SUB)],
                    chunk_v.at[pl.ds(s * SUB, SUB)], sem)
                for s in range(NSUB)
            ]
            for cp in copies:
                cp.wait()
            if p == 1:
                for k in range((NU - TLO) // L):
                    chunk_v[pl.ds(CHA + k * L, L)] = tail_v[pl.ds(k * L, L)]
            for hh in range(NH):
                hb = hh * BH
                pltpu.sync_copy(uid_hbm.at[pl.ds(hb, BH)], uidh_v)

                UNR = 4

                def it(i, p=p, hb=hb, lo=lo):
                    for j in range(UNR):
                        sl = pl.ds((i * UNR + j) * L, L)
                        u = uidh_v[sl]
                        osl = pl.ds(hb + (i * UNR + j) * L, L)
                        if p == 0:
                            m = u < SPLIT
                            g = plsc.load_gather(chunk_v, [u], mask=m)
                            out_v[osl] = jnp.where(m, g, jnp.zeros_like(g))
                        else:
                            m = u >= SPLIT
                            g = plsc.load_gather(chunk_v, [u - lo], mask=m)
                            out_v[osl] = jnp.where(m, g, out_v[osl])

                pl.loop(0, BH // (L * UNR))(it)
        pltpu.sync_copy(out_v, uvT_hbm.at[c])

    return _sc_gather


def _mlp_body(acc_ref, p_ref, e_ref, i_ref, mv_ref, w1t_ref, b1_ref,
              w2t_ref, b2_ref, out_ref):
    del acc_ref
    s0 = lax.rsqrt(mv_ref[3] + 1e-7)
    s1 = lax.rsqrt(mv_ref[4] + 1e-7)
    s2 = lax.rsqrt(mv_ref[5] + 1e-7)
    pn = (p_ref[:] - mv_ref[0]) * s0
    en = (e_ref[:] - mv_ref[1]) * s1
    inn = (i_ref[:] - mv_ref[2]) * s2
    hT = (w1t_ref[:, 0:1] * pn + w1t_ref[:, 1:2] * en
          + w1t_ref[:, 2:3] * inn + b1_ref[:])
    hT = jnp.maximum(hT, 0.0)
    bioT = jnp.dot(w2t_ref[:], hT, preferred_element_type=jnp.float32)
    out_ref[:] = bioT + b2_ref[:]


_BLK = 2048


def _tc_mlp(outT0, P, E, I, mv, W1T, b1c, W2T, b2c):
    return pl.pallas_call(
        _mlp_body,
        grid=(B // _BLK,),
        in_specs=[
            pl.BlockSpec(memory_space=pl.ANY),
            pl.BlockSpec((_BLK,), lambda i: (i,)),
            pl.BlockSpec((_BLK,), lambda i: (i,)),
            pl.BlockSpec((_BLK,), lambda i: (i,)),
            pl.BlockSpec(memory_space=pltpu.SMEM),
            pl.BlockSpec((H, 3), lambda i: (0, 0)),
            pl.BlockSpec((H, 1), lambda i: (0, 0)),
            pl.BlockSpec((D, H), lambda i: (0, 0)),
            pl.BlockSpec((D, 1), lambda i: (0, 0)),
        ],
        out_specs=pl.BlockSpec((D, _BLK), lambda i: (1, i)),
        out_shape=jax.ShapeDtypeStruct((2 * D, B), jnp.float32),
        input_output_aliases={0: 0},
    )(outT0, P, E, I, mv, W1T, b1c, W2T, b2c)


def kernel(user_id, P, E, I, emb_table, norm_mean, norm_var, W1, b1, W2, b2):
    tableT = emb_table.T
    outT0 = _make_sc_gather()(user_id, tableT)
    mv = jnp.concatenate([norm_mean, norm_var])
    outT = _tc_mlp(outT0, P, E, I, mv,
                   W1.T, b1.reshape(H, 1), W2.T, b2.reshape(D, 1))
    return outT.T


# speedup vs baseline: 2.8170x; 1.0306x over previous
import functools

import jax
import jax.numpy as jnp
from jax import lax
from jax.experimental import pallas as pl
from jax.experimental.pallas import tpu as pltpu
from jax.experimental.pallas import tpu_sc as plsc

B = 16384
D = 32
H = 64
NU = 200000
NC = 2
NS = 16
CPS = D // NC
L = 16
CH = 102784
CHA = 102400
SPLIT = 102400
LO2 = 97280
TLO = 199680
NH = 2
BH = B // NH


@functools.cache
def _make_sc_gather():
    mesh = plsc.VectorSubcoreMesh(core_axis_name="c", subcore_axis_name="s")

    @functools.partial(
        pl.kernel,
        mesh=mesh,
        out_type=jax.ShapeDtypeStruct((2 * D, B), jnp.float32),
        scratch_types=[
            pltpu.VMEM((CH,), jnp.float32),
            pltpu.VMEM((NU - TLO,), jnp.float32),
            pltpu.VMEM((BH,), jnp.int32),
            pltpu.VMEM((B,), jnp.float32),
            pltpu.SemaphoreType.DMA,
        ],
        compiler_params=pltpu.CompilerParams(needs_layout_passes=False),
    )
    def _sc_gather(uid_hbm, tableT_hbm, uvT_hbm, chunk_v, tail_v, uidh_v,
                   out_v, sem):
        cid = lax.axis_index("c")
        sid = lax.axis_index("s")
        c = cid * CPS + sid
        pltpu.sync_copy(tableT_hbm.at[c, pl.ds(TLO, NU - TLO)], tail_v)
        NSUB = 4
        SUB = CHA // NSUB
        for p, lo in enumerate((0, LO2)):
            copies = [
                pltpu.async_copy(
                    tableT_hbm.at[c, pl.ds(lo + s * SUB, SUB)],
                    chunk_v.at[pl.ds(s * SUB, SUB)], sem)
                for s in range(NSUB)
            ]
            for cp in copies:
                cp.wait()
            if p == 1:
                for k in range((NU - TLO) // L):
                    chunk_v[pl.ds(CHA + k * L, L)] = tail_v[pl.ds(k * L, L)]
            for hh in range(NH):
                hb = hh * BH
                pltpu.sync_copy(uid_hbm.at[pl.ds(hb, BH)], uidh_v)

                UNR = 4

                def it(i, p=p, hb=hb, lo=lo):
                    for j in range(UNR):
                        sl = pl.ds((i * UNR + j) * L, L)
                        u = uidh_v[sl]
                        osl = pl.ds(hb + (i * UNR + j) * L, L)
                        if p == 0:
                            m = u < SPLIT
                            g = plsc.load_gather(chunk_v, [u], mask=m)
                            out_v[osl] = jnp.where(m, g, jnp.zeros_like(g))
                        else:
                            m = u >= SPLIT
                            g = plsc.load_gather(chunk_v, [u - lo], mask=m)
                            out_v[osl] = jnp.where(m, g, out_v[osl])

                pl.loop(0, BH // (L * UNR))(it)
        pltpu.sync_copy(out_v, uvT_hbm.at[c])

    return _sc_gather


def _mlp_body(acc_ref, p_ref, e_ref, i_ref, mv_ref, w1t_ref, b1_ref,
              w2t_ref, b2_ref, out_ref):
    del acc_ref
    s0 = lax.rsqrt(mv_ref[3] + 1e-7)
    s1 = lax.rsqrt(mv_ref[4] + 1e-7)
    s2 = lax.rsqrt(mv_ref[5] + 1e-7)
    pn = (p_ref[:] - mv_ref[0]) * s0
    en = (e_ref[:] - mv_ref[1]) * s1
    inn = (i_ref[:] - mv_ref[2]) * s2
    hT = (w1t_ref[:, 0:1] * pn + w1t_ref[:, 1:2] * en
          + w1t_ref[:, 2:3] * inn + b1_ref[:])
    hT = jnp.maximum(hT, 0.0)
    bioT = jnp.dot(w2t_ref[:], hT, preferred_element_type=jnp.float32)
    out_ref[:] = bioT + b2_ref[:]


_BLK = 4096


def _tc_mlp(outT0, P, E, I, mv, W1T, b1c, W2T, b2c):
    return pl.pallas_call(
        _mlp_body,
        grid=(B // _BLK,),
        in_specs=[
            pl.BlockSpec(memory_space=pl.ANY),
            pl.BlockSpec((_BLK,), lambda i: (i,)),
            pl.BlockSpec((_BLK,), lambda i: (i,)),
            pl.BlockSpec((_BLK,), lambda i: (i,)),
            pl.BlockSpec(memory_space=pltpu.SMEM),
            pl.BlockSpec((H, 3), lambda i: (0, 0)),
            pl.BlockSpec((H, 1), lambda i: (0, 0)),
            pl.BlockSpec((D, H), lambda i: (0, 0)),
            pl.BlockSpec((D, 1), lambda i: (0, 0)),
        ],
        out_specs=pl.BlockSpec((D, _BLK), lambda i: (1, i)),
        out_shape=jax.ShapeDtypeStruct((2 * D, B), jnp.float32),
        input_output_aliases={0: 0},
    )(outT0, P, E, I, mv, W1T, b1c, W2T, b2c)


def kernel(user_id, P, E, I, emb_table, norm_mean, norm_var, W1, b1, W2, b2):
    tableT = emb_table.T
    outT0 = _make_sc_gather()(user_id, tableT)
    mv = jnp.concatenate([norm_mean, norm_var])
    outT = _tc_mlp(outT0, P, E, I, mv,
                   W1.T, b1.reshape(H, 1), W2.T, b2.reshape(D, 1))
    return outT.T


# speedup vs baseline: 2.8537x; 1.0130x over previous
import functools

import jax
import jax.numpy as jnp
from jax import lax
from jax.experimental import pallas as pl
from jax.experimental.pallas import tpu as pltpu
from jax.experimental.pallas import tpu_sc as plsc

B = 16384
D = 32
H = 64
NU = 200000
NC = 2
NS = 16
CPS = D // NC
L = 16
CH = 102784
CHA = 102400
SPLIT = 102400
LO2 = 97280
TLO = 199680
NH = 2
BH = B // NH


@functools.cache
def _make_sc_gather():
    mesh = plsc.VectorSubcoreMesh(core_axis_name="c", subcore_axis_name="s")

    @functools.partial(
        pl.kernel,
        mesh=mesh,
        out_type=jax.ShapeDtypeStruct((2 * D, B), jnp.float32),
        scratch_types=[
            pltpu.VMEM((CH,), jnp.float32),
            pltpu.VMEM((NU - TLO,), jnp.float32),
            pltpu.VMEM((BH,), jnp.int32),
            pltpu.VMEM((B,), jnp.float32),
            pltpu.SemaphoreType.DMA,
        ],
        compiler_params=pltpu.CompilerParams(needs_layout_passes=False),
    )
    def _sc_gather(uid_hbm, tableT_hbm, uvT_hbm, chunk_v, tail_v, uidh_v,
                   out_v, sem):
        cid = lax.axis_index("c")
        sid = lax.axis_index("s")
        c = cid * CPS + sid
        pltpu.sync_copy(tableT_hbm.at[c, pl.ds(TLO, NU - TLO)], tail_v)
        NSUB = 4
        SUB = CHA // NSUB
        for p, lo in enumerate((0, LO2)):
            copies = [
                pltpu.async_copy(
                    tableT_hbm.at[c, pl.ds(lo + s * SUB, SUB)],
                    chunk_v.at[pl.ds(s * SUB, SUB)], sem)
                for s in range(NSUB)
            ]
            for cp in copies:
                cp.wait()
            if p == 1:
                for k in range((NU - TLO) // L):
                    chunk_v[pl.ds(CHA + k * L, L)] = tail_v[pl.ds(k * L, L)]
            for hh in range(NH):
                hb = hh * BH
                pltpu.sync_copy(uid_hbm.at[pl.ds(hb, BH)], uidh_v)

                UNR = 4

                def it(i, p=p, hb=hb, lo=lo):
                    for j in range(UNR):
                        sl = pl.ds((i * UNR + j) * L, L)
                        u = uidh_v[sl]
                        osl = pl.ds(hb + (i * UNR + j) * L, L)
                        if p == 0:
                            m = u < SPLIT
                            g = plsc.load_gather(chunk_v, [u], mask=m)
                            out_v[osl] = jnp.where(m, g, jnp.zeros_like(g))
                        else:
                            m = u >= SPLIT
                            g = plsc.load_gather(chunk_v, [u - lo], mask=m)
                            out_v[osl] = jnp.where(m, g, out_v[osl])

                pl.loop(0, BH // (L * UNR))(it)
        pltpu.sync_copy(out_v, uvT_hbm.at[c])

    return _sc_gather


def _mlp_body(acc_ref, p_ref, e_ref, i_ref, mv_ref, w1t_ref, b1_ref,
              w2t_ref, b2_ref, out_ref):
    del acc_ref
    s0 = lax.rsqrt(mv_ref[3] + 1e-7)
    s1 = lax.rsqrt(mv_ref[4] + 1e-7)
    s2 = lax.rsqrt(mv_ref[5] + 1e-7)
    pn = (p_ref[:] - mv_ref[0]) * s0
    en = (e_ref[:] - mv_ref[1]) * s1
    inn = (i_ref[:] - mv_ref[2]) * s2
    hT = (w1t_ref[:, 0:1] * pn + w1t_ref[:, 1:2] * en
          + w1t_ref[:, 2:3] * inn + b1_ref[:])
    hT = jnp.maximum(hT, 0.0)
    bioT = jnp.dot(w2t_ref[:], hT, preferred_element_type=jnp.float32)
    out_ref[:] = bioT + b2_ref[:]


_BLK = 8192


def _tc_mlp(outT0, P, E, I, mv, W1T, b1c, W2T, b2c):
    return pl.pallas_call(
        _mlp_body,
        grid=(B // _BLK,),
        in_specs=[
            pl.BlockSpec(memory_space=pl.ANY),
            pl.BlockSpec((_BLK,), lambda i: (i,)),
            pl.BlockSpec((_BLK,), lambda i: (i,)),
            pl.BlockSpec((_BLK,), lambda i: (i,)),
            pl.BlockSpec(memory_space=pltpu.SMEM),
            pl.BlockSpec((H, 3), lambda i: (0, 0)),
            pl.BlockSpec((H, 1), lambda i: (0, 0)),
            pl.BlockSpec((D, H), lambda i: (0, 0)),
            pl.BlockSpec((D, 1), lambda i: (0, 0)),
        ],
        out_specs=pl.BlockSpec((D, _BLK), lambda i: (1, i)),
        out_shape=jax.ShapeDtypeStruct((2 * D, B), jnp.float32),
        input_output_aliases={0: 0},
    )(outT0, P, E, I, mv, W1T, b1c, W2T, b2c)


def kernel(user_id, P, E, I, emb_table, norm_mean, norm_var, W1, b1, W2, b2):
    tableT = emb_table.T
    outT0 = _make_sc_gather()(user_id, tableT)
    mv = jnp.concatenate([norm_mean, norm_var])
    outT = _tc_mlp(outT0, P, E, I, mv,
                   W1.T, b1.reshape(H, 1), W2.T, b2.reshape(D, 1))
    return outT.T


# speedup vs baseline: 2.8589x; 1.0018x over previous
import functools

import jax
import jax.numpy as jnp
from jax import lax
from jax.experimental import pallas as pl
from jax.experimental.pallas import tpu as pltpu
from jax.experimental.pallas import tpu_sc as plsc

B = 16384
D = 32
H = 64
NU = 200000
NC = 2
NS = 16
CPS = D // NC
L = 16
CH = 102784
CHA = 102400
SPLIT = 102400
LO2 = 97280
TLO = 199680
NH = 2
BH = B // NH


@functools.cache
def _make_sc_gather():
    mesh = plsc.VectorSubcoreMesh(core_axis_name="c", subcore_axis_name="s")

    @functools.partial(
        pl.kernel,
        mesh=mesh,
        out_type=jax.ShapeDtypeStruct((2 * D, B), jnp.float32),
        scratch_types=[
            pltpu.VMEM((CH,), jnp.float32),
            pltpu.VMEM((NU - TLO,), jnp.float32),
            pltpu.VMEM((BH,), jnp.int32),
            pltpu.VMEM((B,), jnp.float32),
            pltpu.SemaphoreType.DMA,
        ],
        compiler_params=pltpu.CompilerParams(needs_layout_passes=False),
    )
    def _sc_gather(uid_hbm, tableT_hbm, uvT_hbm, chunk_v, tail_v, uidh_v,
                   out_v, sem):
        cid = lax.axis_index("c")
        sid = lax.axis_index("s")
        c = cid * CPS + sid
        pltpu.sync_copy(tableT_hbm.at[c, pl.ds(TLO, NU - TLO)], tail_v)
        NSUB = 4
        SUB = CHA // NSUB
        for p, lo in enumerate((0, LO2)):
            copies = [
                pltpu.async_copy(
                    tableT_hbm.at[c, pl.ds(lo + s * SUB, SUB)],
                    chunk_v.at[pl.ds(s * SUB, SUB)], sem)
                for s in range(NSUB)
            ]
            for cp in copies:
                cp.wait()
            if p == 1:
                for k in range((NU - TLO) // L):
                    chunk_v[pl.ds(CHA + k * L, L)] = tail_v[pl.ds(k * L, L)]
            for hh in range(NH):
                hb = hh * BH
                pltpu.sync_copy(uid_hbm.at[pl.ds(hb, BH)], uidh_v)

                UNR = 4

                def it(i, p=p, hb=hb, lo=lo):
                    for j in range(UNR):
                        sl = pl.ds((i * UNR + j) * L, L)
                        u = uidh_v[sl]
                        osl = pl.ds(hb + (i * UNR + j) * L, L)
                        if p == 0:
                            m = u < SPLIT
                            g = plsc.load_gather(chunk_v, [u], mask=m)
                            out_v[osl] = jnp.where(m, g, jnp.zeros_like(g))
                        else:
                            m = u >= SPLIT
                            g = plsc.load_gather(chunk_v, [u - lo], mask=m)
                            out_v[osl] = jnp.where(m, g, out_v[osl])

                pl.loop(0, BH // (L * UNR))(it)
        pltpu.sync_copy(out_v, uvT_hbm.at[c])

    return _sc_gather


def _mlp_body(acc_ref, p_ref, e_ref, i_ref, mv_ref, w1t_ref, b1_ref,
              w2t_ref, b2_ref, out_ref):
    del acc_ref
    s0 = lax.rsqrt(mv_ref[3] + 1e-7)
    s1 = lax.rsqrt(mv_ref[4] + 1e-7)
    s2 = lax.rsqrt(mv_ref[5] + 1e-7)
    pn = (p_ref[:] - mv_ref[0]) * s0
    en = (e_ref[:] - mv_ref[1]) * s1
    inn = (i_ref[:] - mv_ref[2]) * s2
    hT = (w1t_ref[:, 0:1] * pn + w1t_ref[:, 1:2] * en
          + w1t_ref[:, 2:3] * inn + b1_ref[:])
    hT = jnp.maximum(hT, 0.0)
    bioT = jnp.dot(w2t_ref[:], hT, preferred_element_type=jnp.float32)
    out_ref[:] = bioT + b2_ref[:]


_BLK = 16384


def _tc_mlp(outT0, P, E, I, mv, W1T, b1c, W2T, b2c):
    return pl.pallas_call(
        _mlp_body,
        grid=(B // _BLK,),
        in_specs=[
            pl.BlockSpec(memory_space=pl.ANY),
            pl.BlockSpec((_BLK,), lambda i: (i,)),
            pl.BlockSpec((_BLK,), lambda i: (i,)),
            pl.BlockSpec((_BLK,), lambda i: (i,)),
            pl.BlockSpec(memory_space=pltpu.SMEM),
            pl.BlockSpec((H, 3), lambda i: (0, 0)),
            pl.BlockSpec((H, 1), lambda i: (0, 0)),
            pl.BlockSpec((D, H), lambda i: (0, 0)),
            pl.BlockSpec((D, 1), lambda i: (0, 0)),
        ],
        out_specs=pl.BlockSpec((D, _BLK), lambda i: (1, i)),
        out_shape=jax.ShapeDtypeStruct((2 * D, B), jnp.float32),
        input_output_aliases={0: 0},
    )(outT0, P, E, I, mv, W1T, b1c, W2T, b2c)


def kernel(user_id, P, E, I, emb_table, norm_mean, norm_var, W1, b1, W2, b2):
    tableT = emb_table.T
    outT0 = _make_sc_gather()(user_id, tableT)
    mv = jnp.concatenate([norm_mean, norm_var])
    outT = _tc_mlp(outT0, P, E, I, mv,
                   W1.T, b1.reshape(H, 1), W2.T, b2.reshape(D, 1))
    return outT.T
